# baseline (speedup 1.0000x reference)
import functools

import jax
import jax.numpy as jnp
from jax import lax
from jax.experimental import pallas as pl
from jax.experimental.pallas import tpu as pltpu
from jax.experimental.pallas import tpu_sc as plsc

_NC = 2
_NS = 16
_NW = _NC * _NS


def _sc_gather(nodes, senders):
    n, d = nodes.shape
    e = senders.shape[0]
    ew = e // _NW
    gk = 200
    nch = ew // gk
    mesh = plsc.VectorSubcoreMesh(core_axis_name="c", subcore_axis_name="s")

    @functools.partial(
        pl.kernel,
        out_type=jax.ShapeDtypeStruct((e, d), jnp.int32),
        mesh=mesh,
        scratch_types=[
            pltpu.VMEM((gk,), jnp.int32),
            pltpu.VMEM((gk,), jnp.int32),
            pltpu.VMEM((gk, d), jnp.int32),
            pltpu.VMEM((gk, d), jnp.int32),
            pltpu.SemaphoreType.DMA,
            pltpu.SemaphoreType.DMA,
            pltpu.SemaphoreType.DMA,
            pltpu.SemaphoreType.DMA,
            pltpu.SemaphoreType.DMA,
            pltpu.SemaphoreType.DMA,
        ],
    )
    def k(nodes_hbm, senders_hbm, out_hbm, idx0_v, idx1_v, rows0_v, rows1_v,
          si0, si1, sg0, sg1, sw0, sw1):
        wid = lax.axis_index("s") * _NC + lax.axis_index("c")
        idx = (idx0_v, idx1_v)
        rows = (rows0_v, rows1_v)
        si = (si0, si1)
        sg = (sg0, sg1)
        sw = (sw0, sw1)

        def base(j):
            return pl.multiple_of(wid * ew + j * gk, 8)

        def load(j):
            b = j % 2
            return pltpu.async_copy(senders_hbm.at[pl.ds(base(j), gk)],
                                    idx[b], si[b])

        def gath(j):
            b = j % 2
            return pltpu.async_copy(nodes_hbm.at[idx[b]],
                                    rows[b], sg[b])

        def wout(j):
            b = j % 2
            return pltpu.async_copy(rows[b],
                                    out_hbm.at[pl.ds(base(j), gk)], sw[b])

        dl = [None] * nch
        dg = [None] * nch
        dw = [None] * nch
        dl[0] = load(0)
        if nch > 1:
            dl[1] = load(1)
        dl[0].wait()
        dg[0] = gath(0)
        for j in range(nch):
            dg[j].wait()
            if j + 1 < nch:
                dl[j + 1].wait()
                if j >= 1:
                    dw[j - 1].wait()
                dg[j + 1] = gath(j + 1)
            dw[j] = wout(j)
            if j + 2 < nch:
                dl[j + 2] = load(j + 2)
        if nch > 1:
            dw[nch - 2].wait()
        dw[nch - 1].wait()

    return k(nodes, senders)


def _sc_scatter(messages, receivers, n):
    e, h = messages.shape
    ew = e // _NW
    sk = 192
    nch = ew // sk
    rem = ew - nch * sk
    cw = 128
    np_ = h // cw
    rps = 632
    rlast = n - (_NS - 1) * rps

    z128 = jnp.zeros((rps, cw), jnp.float32)
    e1 = jnp.zeros((sk, cw), jnp.float32).at[:, 0].set(1.0)

    mesh = plsc.VectorSubcoreMesh(core_axis_name="c", subcore_axis_name="s")

    @functools.partial(
        pl.kernel,
        out_type=(
            jax.ShapeDtypeStruct((_NC, n, h), jnp.float32),
            jax.ShapeDtypeStruct((_NC, n, cw), jnp.float32),
        ),
        mesh=mesh,
        scratch_types=[
            pltpu.VMEM((sk, cw), jnp.float32),
            pltpu.VMEM((sk, cw), jnp.float32),
            pltpu.VMEM((rem, cw), jnp.float32),
            pltpu.VMEM((sk,), jnp.int32),
            pltpu.VMEM((sk,), jnp.int32),
            pltpu.VMEM((rem,), jnp.int32),
            pltpu.VMEM_SHARED((n, cw), jnp.float32),
            pltpu.SemaphoreType.DMA,
            pltpu.SemaphoreType.DMA,
            pltpu.SemaphoreType.DMA,
            pltpu.SemaphoreType.DMA,
            pltpu.SemaphoreType.DMA,
        ],
    )
    def k(msgs_hbm, recv_hbm, z128_hbm, e1_hbm, agg_out, cnt_out,
          msg0_v, msg1_v, msge_v, idx0_v, idx1_v, idxe_v, agg_s,
          si0, si1, sm0, sm1, se):
        c = lax.axis_index("c")
        s = lax.axis_index("s")
        wid = s * _NC + c
        row0 = pl.multiple_of(s * rps, 8)
        is_last = s == _NS - 1
        wbase = wid * ew
        msgb = (msg0_v, msg1_v)
        idxb = (idx0_v, idx1_v)
        sib = (si0, si1)
        smb = (sm0, sm1)

        def fire(j, b, p):
            e0 = pl.multiple_of(wbase + j * sk, 8)
            pltpu.async_copy(recv_hbm.at[pl.ds(e0, sk)], idxb[b], sib[b])
            if p < np_:
                pltpu.async_copy(
                    msgs_hbm.at[pl.ds(e0, sk), pl.ds(p * cw, cw)],
                    msgb[b], smb[b])

        def waitld(j, b, p):
            e0 = pl.multiple_of(wbase + j * sk, 8)
            pltpu.make_async_copy(recv_hbm.at[pl.ds(e0, sk)],
                                  idxb[b], sib[b]).wait()
            if p < np_:
                pltpu.make_async_copy(
                    msgs_hbm.at[pl.ds(e0, sk), pl.ds(p * cw, cw)],
                    msgb[b], smb[b]).wait()

        for p in range(np_ + 1):
            @pl.when(jnp.logical_not(is_last))
            def _():
                pltpu.sync_copy(z128_hbm, agg_s.at[pl.ds(row0, rps)])

            @pl.when(is_last)
            def _():
                pltpu.sync_copy(z128_hbm.at[pl.ds(0, rlast)],
                                agg_s.at[pl.ds(row0, rlast)])

            plsc.subcore_barrier()

            if p == np_:
                pltpu.sync_copy(e1_hbm, msg0_v)
                pltpu.sync_copy(e1_hbm.at[pl.ds(0, rem)], msge_v)

            fire(0, 0, p)

            def super(i, carry):
                j0 = 2 * i
                fire(j0 + 1, 1, p)
                waitld(j0, 0, p)
                pltpu.sync_copy(msg0_v, agg_s.at[idx0_v], add=True)

                @pl.when(i < nch // 2 - 1)
                def _():
                    fire(j0 + 2, 0, p)

                waitld(j0 + 1, 1, p)
                pltpu.sync_copy(msg1_v if p < np_ else msg0_v,
                                agg_s.at[idx1_v], add=True)
                return carry

            lax.fori_loop(0, nch // 2, super, 0)

            if rem:
                e0r = pl.multiple_of(wbase + nch * sk, 8)
                pltpu.async_copy(recv_hbm.at[pl.ds(e0r, rem)], idxe_v,
                                 se).wait()
                if p < np_:
                    pltpu.async_copy(
                        msgs_hbm.at[pl.ds(e0r, rem), pl.ds(p * cw, cw)],
                        msge_v, se).wait()
                pltpu.sync_copy(msge_v, agg_s.at[idxe_v], add=True)
            plsc.subcore_barrier()

            @pl.when(jnp.logical_not(is_last))
            def _():
                if p < np_:
                    pltpu.sync_copy(
                        agg_s.at[pl.ds(row0, rps)],
                        agg_out.at[c, pl.ds(row0, rps), pl.ds(p * cw, cw)])
                else:
                    pltpu.sync_copy(agg_s.at[pl.ds(row0, rps)],
                                    cnt_out.at[c, pl.ds(row0, rps)])

            @pl.when(is_last)
            def _():
                if p < np_:
                    pltpu.sync_copy(
                        agg_s.at[pl.ds(row0, rlast)],
                        agg_out.at[c, pl.ds(row0, rlast), pl.ds(p * cw, cw)])
                else:
                    pltpu.sync_copy(agg_s.at[pl.ds(row0, rlast)],
                                    cnt_out.at[c, pl.ds(row0, rlast)])

            plsc.subcore_barrier()

    return k(messages, receivers, z128, e1)


def _layer_norm_in_kernel(h, g, b):
    mu = jnp.mean(h, axis=-1, keepdims=True)
    var = jnp.mean((h - mu) * (h - mu), axis=-1, keepdims=True)
    return (h - mu) * lax.rsqrt(var + 1e-6) * g + b


def _tc_edge_mlp(sf, edges, w1t, w1b, b1, g1, be1, w2, b2):
    e, d = sf.shape
    de = edges.shape[1]
    hdim = w2.shape[1]
    be_blk = 1280
    grid = (e // be_blk,)

    def body(sf_ref, ed_ref, w1t_ref, w1b_ref, b1_ref, g1_ref, be1_ref,
             w2_ref, b2_ref, out_ref):
        h = jnp.dot(sf_ref[...], w1t_ref[...],
                    preferred_element_type=jnp.float32)
        h = h + jnp.dot(ed_ref[...].astype(jnp.bfloat16), w1b_ref[...],
                        preferred_element_type=jnp.float32)
        h = h + b1_ref[...]
        h = _layer_norm_in_kernel(h, g1_ref[...], be1_ref[...])
        h = jnp.maximum(h, 0.0)
        out_ref[...] = jnp.dot(h.astype(jnp.bfloat16), w2_ref[...],
                               preferred_element_type=jnp.float32) + b2_ref[...]

    hsz = w1t.shape[1]
    return pl.pallas_call(
        body,
        grid=grid,
        in_specs=[
            pl.BlockSpec((be_blk, d), lambda i: (i, 0)),
            pl.BlockSpec((be_blk, de), lambda i: (i, 0)),
            pl.BlockSpec((d, hsz), lambda i: (0, 0)),
            pl.BlockSpec((de, hsz), lambda i: (0, 0)),
            pl.BlockSpec((1, hsz), lambda i: (0, 0)),
            pl.BlockSpec((1, hsz), lambda i: (0, 0)),
            pl.BlockSpec((1, hsz), lambda i: (0, 0)),
            pl.BlockSpec((hsz, hdim), lambda i: (0, 0)),
            pl.BlockSpec((1, hdim), lambda i: (0, 0)),
        ],
        out_specs=pl.BlockSpec((be_blk, hdim), lambda i: (i, 0)),
        out_shape=jax.ShapeDtypeStruct((e, hdim), jnp.float32),
    )(sf, edges, w1t, w1b, b1, g1, be1, w2, b2)


def _tc_node_mlp(nodes, a0, a1, c0, c1, w1t, w1b, b1, g1, be1, w2, b2):
    n, d = nodes.shape
    hdim = a0.shape[1]
    do = w2.shape[1]
    bn = 1000
    grid = (n // bn,)

    def body(nd_ref, a0_ref, a1_ref, c0_ref, c1_ref, w1t_ref, w1b_ref,
             b1_ref, g1_ref, be1_ref, w2_ref, b2_ref, out_ref):
        cnt = (c0_ref[...][:, 0:1] + c1_ref[...][:, 0:1]).astype(jnp.float32)
        cnt = jnp.maximum(cnt, 1.0)
        agg = (a0_ref[...].astype(jnp.float32)
               + a1_ref[...].astype(jnp.float32)) / cnt
        h = jnp.dot(nd_ref[...].astype(jnp.bfloat16), w1t_ref[...],
                    preferred_element_type=jnp.float32)
        h = h + jnp.dot(agg.astype(jnp.bfloat16), w1b_ref[...],
                        preferred_element_type=jnp.float32)
        h = h + b1_ref[...]
        h = _layer_norm_in_kernel(h, g1_ref[...], be1_ref[...])
        h = jnp.maximum(h, 0.0)
        out = jnp.dot(h.astype(jnp.bfloat16), w2_ref[...],
                      preferred_element_type=jnp.float32) + b2_ref[...]
        out_ref[...] = out + nd_ref[...]

    hsz = w1t.shape[1]
    return pl.pallas_call(
        body,
        grid=grid,
        in_specs=[
            pl.BlockSpec((bn, d), lambda i: (i, 0)),
            pl.BlockSpec((bn, hdim), lambda i: (i, 0)),
            pl.BlockSpec((bn, hdim), lambda i: (i, 0)),
            pl.BlockSpec((bn, 128), lambda i: (i, 0)),
            pl.BlockSpec((bn, 128), lambda i: (i, 0)),
            pl.BlockSpec((d, hsz), lambda i: (0, 0)),
            pl.BlockSpec((hdim, hsz), lambda i: (0, 0)),
            pl.BlockSpec((1, hsz), lambda i: (0, 0)),
            pl.BlockSpec((1, hsz), lambda i: (0, 0)),
            pl.BlockSpec((1, hsz), lambda i: (0, 0)),
            pl.BlockSpec((hsz, do), lambda i: (0, 0)),
            pl.BlockSpec((1, do), lambda i: (0, 0)),
        ],
        out_specs=pl.BlockSpec((bn, do), lambda i: (i, 0)),
        out_shape=jax.ShapeDtypeStruct((n, do), jnp.float32),
    )(nodes, a0, a1, c0, c1, w1t, w1b, b1, g1, be1, w2, b2)


def kernel(nodes, edges, senders, receivers, W1e, b1e, g1e, be1e, W2e, b2e,
           W1n, b1n, g1n, be1n, W2n, b2n):
    n, df = nodes.shape
    e, de = edges.shape
    senders = senders.astype(jnp.int32)
    receivers = receivers.astype(jnp.int32)

    nodes_packed = lax.bitcast_convert_type(
        nodes.astype(jnp.bfloat16).reshape(n, df // 2, 2), jnp.int32)
    sf_packed = _sc_gather(nodes_packed, senders)
    sf = lax.bitcast_convert_type(sf_packed, jnp.bfloat16).reshape(e, df)
    messages = _tc_edge_mlp(
        sf, edges,
        W1e[:df].astype(jnp.bfloat16), W1e[df:].astype(jnp.bfloat16),
        b1e[None, :], g1e[None, :], be1e[None, :],
        W2e.astype(jnp.bfloat16), b2e[None, :])
    agg_part, cnt_part = _sc_scatter(messages, receivers, n)
    new_nodes = _tc_node_mlp(
        nodes, agg_part[0], agg_part[1], cnt_part[0], cnt_part[1],
        W1n[:df].astype(jnp.bfloat16), W1n[df:].astype(jnp.bfloat16),
        b1n[None, :], g1n[None, :], be1n[None, :],
        W2n.astype(jnp.bfloat16), b2n[None, :])
    return new_nodes


# speedup vs baseline: 1.5848x; 1.5848x over previous
import functools

import jax
import jax.numpy as jnp
from jax import lax
from jax.experimental import pallas as pl
from jax.experimental.pallas import tpu as pltpu
from jax.experimental.pallas import tpu_sc as plsc

_NC = 2
_NS = 16
_NW = _NC * _NS


def _sc_gather(nodes, senders):
    n, d = nodes.shape
    e = senders.shape[0]
    ew = e // _NW
    gk = 200
    nch = ew // gk
    mesh = plsc.VectorSubcoreMesh(core_axis_name="c", subcore_axis_name="s")

    @functools.partial(
        pl.kernel,
        out_type=jax.ShapeDtypeStruct((e, d), jnp.int32),
        mesh=mesh,
        scratch_types=[
            pltpu.VMEM((gk,), jnp.int32),
            pltpu.VMEM((gk,), jnp.int32),
            pltpu.VMEM((gk, d), jnp.int32),
            pltpu.VMEM((gk, d), jnp.int32),
            pltpu.SemaphoreType.DMA,
            pltpu.SemaphoreType.DMA,
            pltpu.SemaphoreType.DMA,
            pltpu.SemaphoreType.DMA,
            pltpu.SemaphoreType.DMA,
            pltpu.SemaphoreType.DMA,
        ],
    )
    def k(nodes_hbm, senders_hbm, out_hbm, idx0_v, idx1_v, rows0_v, rows1_v,
          si0, si1, sg0, sg1, sw0, sw1):
        wid = lax.axis_index("s") * _NC + lax.axis_index("c")
        idx = (idx0_v, idx1_v)
        rows = (rows0_v, rows1_v)
        si = (si0, si1)
        sg = (sg0, sg1)
        sw = (sw0, sw1)

        def base(j):
            return pl.multiple_of(wid * ew + j * gk, 8)

        def load(j):
            b = j % 2
            return pltpu.async_copy(senders_hbm.at[pl.ds(base(j), gk)],
                                    idx[b], si[b])

        def gath(j):
            b = j % 2
            return pltpu.async_copy(nodes_hbm.at[idx[b]],
                                    rows[b], sg[b])

        def wout(j):
            b = j % 2
            return pltpu.async_copy(rows[b],
                                    out_hbm.at[pl.ds(base(j), gk)], sw[b])

        dl = [None] * nch
        dg = [None] * nch
        dw = [None] * nch
        dl[0] = load(0)
        if nch > 1:
            dl[1] = load(1)
        dl[0].wait()
        dg[0] = gath(0)
        for j in range(nch):
            dg[j].wait()
            if j + 1 < nch:
                dl[j + 1].wait()
                if j >= 1:
                    dw[j - 1].wait()
                dg[j + 1] = gath(j + 1)
            dw[j] = wout(j)
            if j + 2 < nch:
                dl[j + 2] = load(j + 2)
        if nch > 1:
            dw[nch - 2].wait()
        dw[nch - 1].wait()

    return k(nodes, senders)


def _sc_scatter(messages, receivers, n):
    e, h = messages.shape
    ew = e // _NW
    sk = 192
    nch = ew // sk
    rem = ew - nch * sk
    cw = 128
    np_ = h // cw
    rps = 632
    rlast = n - (_NS - 1) * rps

    z128 = jnp.zeros((rps, cw), jnp.float32)
    e1 = jnp.zeros((sk, cw), jnp.float32).at[:, 0].set(1.0)

    mesh = plsc.VectorSubcoreMesh(core_axis_name="c", subcore_axis_name="s")

    @functools.partial(
        pl.kernel,
        out_type=(
            jax.ShapeDtypeStruct((_NC, n, h), jnp.float32),
            jax.ShapeDtypeStruct((_NC, n, cw), jnp.float32),
        ),
        mesh=mesh,
        scratch_types=[
            pltpu.VMEM((sk, cw), jnp.float32),
            pltpu.VMEM((sk, cw), jnp.float32),
            pltpu.VMEM((rem, cw), jnp.float32),
            pltpu.VMEM((sk,), jnp.int32),
            pltpu.VMEM((sk,), jnp.int32),
            pltpu.VMEM((rem,), jnp.int32),
            pltpu.VMEM_SHARED((n, cw), jnp.float32),
            pltpu.SemaphoreType.DMA,
            pltpu.SemaphoreType.DMA,
            pltpu.SemaphoreType.DMA,
            pltpu.SemaphoreType.DMA,
            pltpu.SemaphoreType.DMA,
        ],
    )
    def k(msgs_hbm, recv_hbm, z128_hbm, e1_hbm, agg_out, cnt_out,
          msg0_v, msg1_v, msge_v, idx0_v, idx1_v, idxe_v, agg_s,
          si0, si1, sm0, sm1, se):
        c = lax.axis_index("c")
        s = lax.axis_index("s")
        wid = s * _NC + c
        row0 = pl.multiple_of(s * rps, 8)
        is_last = s == _NS - 1
        wbase = wid * ew
        msgb = (msg0_v, msg1_v)
        idxb = (idx0_v, idx1_v)
        sib = (si0, si1)
        smb = (sm0, sm1)

        def fire(j, b, p):
            e0 = pl.multiple_of(wbase + j * sk, 8)
            pltpu.async_copy(recv_hbm.at[pl.ds(e0, sk)], idxb[b], sib[b])
            if p < np_:
                pltpu.async_copy(
                    msgs_hbm.at[pl.ds(e0, sk), pl.ds(p * cw, cw)],
                    msgb[b], smb[b])

        def waitld(j, b, p):
            e0 = pl.multiple_of(wbase + j * sk, 8)
            pltpu.make_async_copy(recv_hbm.at[pl.ds(e0, sk)],
                                  idxb[b], sib[b]).wait()
            if p < np_:
                pltpu.make_async_copy(
                    msgs_hbm.at[pl.ds(e0, sk), pl.ds(p * cw, cw)],
                    msgb[b], smb[b]).wait()

        for p in range(np_ + 1):
            @pl.when(jnp.logical_not(is_last))
            def _():
                pltpu.sync_copy(z128_hbm, agg_s.at[pl.ds(row0, rps)])

            @pl.when(is_last)
            def _():
                pltpu.sync_copy(z128_hbm.at[pl.ds(0, rlast)],
                                agg_s.at[pl.ds(row0, rlast)])

            plsc.subcore_barrier()

            if p == np_:
                pltpu.sync_copy(e1_hbm, msg0_v)
                pltpu.sync_copy(e1_hbm.at[pl.ds(0, rem)], msge_v)

            fire(0, 0, p)

            def super(i, carry):
                j0 = 2 * i
                fire(j0 + 1, 1, p)
                waitld(j0, 0, p)
                pltpu.sync_copy(msg0_v, agg_s.at[idx0_v], add=True)

                @pl.when(i < nch // 2 - 1)
                def _():
                    fire(j0 + 2, 0, p)

                waitld(j0 + 1, 1, p)
                pltpu.sync_copy(msg1_v if p < np_ else msg0_v,
                                agg_s.at[idx1_v], add=True)
                return carry

            lax.fori_loop(0, nch // 2, super, 0)

            if rem:
                e0r = pl.multiple_of(wbase + nch * sk, 8)
                pltpu.async_copy(recv_hbm.at[pl.ds(e0r, rem)], idxe_v,
                                 se).wait()
                if p < np_:
                    pltpu.async_copy(
                        msgs_hbm.at[pl.ds(e0r, rem), pl.ds(p * cw, cw)],
                        msge_v, se).wait()
                pltpu.sync_copy(msge_v, agg_s.at[idxe_v], add=True)
            plsc.subcore_barrier()

            @pl.when(jnp.logical_not(is_last))
            def _():
                if p < np_:
                    pltpu.sync_copy(
                        agg_s.at[pl.ds(row0, rps)],
                        agg_out.at[c, pl.ds(row0, rps), pl.ds(p * cw, cw)])
                else:
                    pltpu.sync_copy(agg_s.at[pl.ds(row0, rps)],
                                    cnt_out.at[c, pl.ds(row0, rps)])

            @pl.when(is_last)
            def _():
                if p < np_:
                    pltpu.sync_copy(
                        agg_s.at[pl.ds(row0, rlast)],
                        agg_out.at[c, pl.ds(row0, rlast), pl.ds(p * cw, cw)])
                else:
                    pltpu.sync_copy(agg_s.at[pl.ds(row0, rlast)],
                                    cnt_out.at[c, pl.ds(row0, rlast)])

            plsc.subcore_barrier()

    return k(messages, receivers, z128, e1)


def _layer_norm_in_kernel(h, g, b):
    mu = jnp.mean(h, axis=-1, keepdims=True)
    var = jnp.mean((h - mu) * (h - mu), axis=-1, keepdims=True)
    return (h - mu) * lax.rsqrt(var + 1e-6) * g + b


def _tc_edge_mlp(sfp, edges, w1lo, w1hi, w1b, b1, g1, be1, w2, b2):
    e, dp = sfp.shape
    de = edges.shape[1]
    hdim = w2.shape[1]
    be_blk = 1280
    grid = (e // be_blk,)

    def body(sf_ref, ed_ref, w1lo_ref, w1hi_ref, w1b_ref, b1_ref, g1_ref,
             be1_ref, w2_ref, b2_ref, out_ref):
        spk = sf_ref[...]
        lo = lax.bitcast_convert_type(
            lax.shift_left(spk, 16), jnp.float32).astype(jnp.bfloat16)
        hi = lax.bitcast_convert_type(
            lax.bitwise_and(spk, jnp.int32(-65536)),
            jnp.float32).astype(jnp.bfloat16)
        h = jnp.dot(lo, w1lo_ref[...], preferred_element_type=jnp.float32)
        h = h + jnp.dot(hi, w1hi_ref[...], preferred_element_type=jnp.float32)
        h = h + jnp.dot(ed_ref[...].astype(jnp.bfloat16), w1b_ref[...],
                        preferred_element_type=jnp.float32)
        h = h + b1_ref[...]
        h = _layer_norm_in_kernel(h, g1_ref[...], be1_ref[...])
        h = jnp.maximum(h, 0.0)
        out_ref[...] = jnp.dot(h.astype(jnp.bfloat16), w2_ref[...],
                               preferred_element_type=jnp.float32) + b2_ref[...]

    hsz = w1lo.shape[1]
    return pl.pallas_call(
        body,
        grid=grid,
        in_specs=[
            pl.BlockSpec((be_blk, dp), lambda i: (i, 0)),
            pl.BlockSpec((be_blk, de), lambda i: (i, 0)),
            pl.BlockSpec((dp, hsz), lambda i: (0, 0)),
            pl.BlockSpec((dp, hsz), lambda i: (0, 0)),
            pl.BlockSpec((de, hsz), lambda i: (0, 0)),
            pl.BlockSpec((1, hsz), lambda i: (0, 0)),
            pl.BlockSpec((1, hsz), lambda i: (0, 0)),
            pl.BlockSpec((1, hsz), lambda i: (0, 0)),
            pl.BlockSpec((hsz, hdim), lambda i: (0, 0)),
            pl.BlockSpec((1, hdim), lambda i: (0, 0)),
        ],
        out_specs=pl.BlockSpec((be_blk, hdim), lambda i: (i, 0)),
        out_shape=jax.ShapeDtypeStruct((e, hdim), jnp.float32),
    )(sfp, edges, w1lo, w1hi, w1b, b1, g1, be1, w2, b2)


def _tc_node_mlp(nodes, a0, a1, c0, c1, w1t, w1b, b1, g1, be1, w2, b2):
    n, d = nodes.shape
    hdim = a0.shape[1]
    do = w2.shape[1]
    bn = 1000
    grid = (n // bn,)

    def body(nd_ref, a0_ref, a1_ref, c0_ref, c1_ref, w1t_ref, w1b_ref,
             b1_ref, g1_ref, be1_ref, w2_ref, b2_ref, out_ref):
        cnt = (c0_ref[...][:, 0:1] + c1_ref[...][:, 0:1]).astype(jnp.float32)
        cnt = jnp.maximum(cnt, 1.0)
        agg = (a0_ref[...].astype(jnp.float32)
               + a1_ref[...].astype(jnp.float32)) / cnt
        h = jnp.dot(nd_ref[...].astype(jnp.bfloat16), w1t_ref[...],
                    preferred_element_type=jnp.float32)
        h = h + jnp.dot(agg.astype(jnp.bfloat16), w1b_ref[...],
                        preferred_element_type=jnp.float32)
        h = h + b1_ref[...]
        h = _layer_norm_in_kernel(h, g1_ref[...], be1_ref[...])
        h = jnp.maximum(h, 0.0)
        out = jnp.dot(h.astype(jnp.bfloat16), w2_ref[...],
                      preferred_element_type=jnp.float32) + b2_ref[...]
        out_ref[...] = out + nd_ref[...]

    hsz = w1t.shape[1]
    return pl.pallas_call(
        body,
        grid=grid,
        in_specs=[
            pl.BlockSpec((bn, d), lambda i: (i, 0)),
            pl.BlockSpec((bn, hdim), lambda i: (i, 0)),
            pl.BlockSpec((bn, hdim), lambda i: (i, 0)),
            pl.BlockSpec((bn, 128), lambda i: (i, 0)),
            pl.BlockSpec((bn, 128), lambda i: (i, 0)),
            pl.BlockSpec((d, hsz), lambda i: (0, 0)),
            pl.BlockSpec((hdim, hsz), lambda i: (0, 0)),
            pl.BlockSpec((1, hsz), lambda i: (0, 0)),
            pl.BlockSpec((1, hsz), lambda i: (0, 0)),
            pl.BlockSpec((1, hsz), lambda i: (0, 0)),
            pl.BlockSpec((hsz, do), lambda i: (0, 0)),
            pl.BlockSpec((1, do), lambda i: (0, 0)),
        ],
        out_specs=pl.BlockSpec((bn, do), lambda i: (i, 0)),
        out_shape=jax.ShapeDtypeStruct((n, do), jnp.float32),
    )(nodes, a0, a1, c0, c1, w1t, w1b, b1, g1, be1, w2, b2)


def kernel(nodes, edges, senders, receivers, W1e, b1e, g1e, be1e, W2e, b2e,
           W1n, b1n, g1n, be1n, W2n, b2n):
    n, df = nodes.shape
    e, de = edges.shape
    senders = senders.astype(jnp.int32)
    receivers = receivers.astype(jnp.int32)

    nodes_packed = lax.bitcast_convert_type(
        nodes.astype(jnp.bfloat16).reshape(n, df // 2, 2), jnp.int32)
    sf_packed = _sc_gather(nodes_packed, senders)
    messages = _tc_edge_mlp(
        sf_packed, edges,
        W1e[0:df:2].astype(jnp.bfloat16), W1e[1:df:2].astype(jnp.bfloat16),
        W1e[df:].astype(jnp.bfloat16),
        b1e[None, :], g1e[None, :], be1e[None, :],
        W2e.astype(jnp.bfloat16), b2e[None, :])
    agg_part, cnt_part = _sc_scatter(messages, receivers, n)
    new_nodes = _tc_node_mlp(
        nodes, agg_part[0], agg_part[1], cnt_part[0], cnt_part[1],
        W1n[:df].astype(jnp.bfloat16), W1n[df:].astype(jnp.bfloat16),
        b1n[None, :], g1n[None, :], be1n[None, :],
        W2n.astype(jnp.bfloat16), b2n[None, :])
    return new_nodes


# speedup vs baseline: 1.8395x; 1.1607x over previous
import functools

import jax
import jax.numpy as jnp
from jax import lax
from jax.experimental import pallas as pl
from jax.experimental.pallas import tpu as pltpu
from jax.experimental.pallas import tpu_sc as plsc

_NC = 2
_NS = 16
_NW = _NC * _NS


def _sc_gather(nodes, senders):
    n, d = nodes.shape
    e = senders.shape[0]
    ew = e // _NW
    gk = 200
    nch = ew // gk
    mesh = plsc.VectorSubcoreMesh(core_axis_name="c", subcore_axis_name="s")

    @functools.partial(
        pl.kernel,
        out_type=jax.ShapeDtypeStruct((e, d), jnp.int32),
        mesh=mesh,
        scratch_types=[
            pltpu.VMEM((gk,), jnp.int32),
            pltpu.VMEM((gk,), jnp.int32),
            pltpu.VMEM((gk, d), jnp.int32),
            pltpu.VMEM((gk, d), jnp.int32),
            pltpu.SemaphoreType.DMA,
            pltpu.SemaphoreType.DMA,
            pltpu.SemaphoreType.DMA,
            pltpu.SemaphoreType.DMA,
            pltpu.SemaphoreType.DMA,
            pltpu.SemaphoreType.DMA,
        ],
    )
    def k(nodes_hbm, senders_hbm, out_hbm, idx0_v, idx1_v, rows0_v, rows1_v,
          si0, si1, sg0, sg1, sw0, sw1):
        wid = lax.axis_index("s") * _NC + lax.axis_index("c")
        idx = (idx0_v, idx1_v)
        rows = (rows0_v, rows1_v)
        si = (si0, si1)
        sg = (sg0, sg1)
        sw = (sw0, sw1)

        def base(j):
            return pl.multiple_of(wid * ew + j * gk, 8)

        def load(j):
            b = j % 2
            return pltpu.async_copy(senders_hbm.at[pl.ds(base(j), gk)],
                                    idx[b], si[b])

        def gath(j):
            b = j % 2
            return pltpu.async_copy(nodes_hbm.at[idx[b]],
                                    rows[b], sg[b])

        def wout(j):
            b = j % 2
            return pltpu.async_copy(rows[b],
                                    out_hbm.at[pl.ds(base(j), gk)], sw[b])

        dl = [None] * nch
        dg = [None] * nch
        dw = [None] * nch
        dl[0] = load(0)
        if nch > 1:
            dl[1] = load(1)
        dl[0].wait()
        dg[0] = gath(0)
        for j in range(nch):
            dg[j].wait()
            if j + 1 < nch:
                dl[j + 1].wait()
                if j >= 1:
                    dw[j - 1].wait()
                dg[j + 1] = gath(j + 1)
            dw[j] = wout(j)
            if j + 2 < nch:
                dl[j + 2] = load(j + 2)
        if nch > 1:
            dw[nch - 2].wait()
        dw[nch - 1].wait()

    return k(nodes, senders)


def _sc_scatter(messages, receivers, n):
    e, h = messages.shape
    ew = e // _NW
    sk = 192
    nch = ew // sk
    rem = ew - nch * sk
    cw = 128
    np_ = h // cw
    rps = 632
    rlast = n - (_NS - 1) * rps

    z128 = jnp.zeros((rps, cw), jnp.float32)
    e1 = jnp.zeros((sk, cw), jnp.float32).at[:, 0].set(1.0)

    mesh = plsc.VectorSubcoreMesh(core_axis_name="c", subcore_axis_name="s")

    @functools.partial(
        pl.kernel,
        out_type=(
            jax.ShapeDtypeStruct((_NC, n, h), jnp.float32),
            jax.ShapeDtypeStruct((_NC, n, cw), jnp.float32),
        ),
        mesh=mesh,
        scratch_types=[
            pltpu.VMEM((sk, cw), jnp.float32),
            pltpu.VMEM((sk, cw), jnp.float32),
            pltpu.VMEM((rem, cw), jnp.float32),
            pltpu.VMEM((sk,), jnp.int32),
            pltpu.VMEM((sk,), jnp.int32),
            pltpu.VMEM((rem,), jnp.int32),
            pltpu.VMEM_SHARED((n, cw), jnp.float32),
            pltpu.SemaphoreType.DMA,
            pltpu.SemaphoreType.DMA,
            pltpu.SemaphoreType.DMA,
            pltpu.SemaphoreType.DMA,
            pltpu.SemaphoreType.DMA,
        ],
    )
    def k(msgs_hbm, recv_hbm, z128_hbm, e1_hbm, agg_out, cnt_out,
          msg0_v, msg1_v, msge_v, idx0_v, idx1_v, idxe_v, agg_s,
          si0, si1, sm0, sm1, se):
        c = lax.axis_index("c")
        s = lax.axis_index("s")
        wid = s * _NC + c
        row0 = pl.multiple_of(s * rps, 8)
        is_last = s == _NS - 1
        wbase = wid * ew
        msgb = (msg0_v, msg1_v)
        idxb = (idx0_v, idx1_v)
        sib = (si0, si1)
        smb = (sm0, sm1)

        def fire(j, b, p):
            e0 = pl.multiple_of(wbase + j * sk, 8)
            pltpu.async_copy(recv_hbm.at[pl.ds(e0, sk)], idxb[b], sib[b])
            if p < np_:
                pltpu.async_copy(
                    msgs_hbm.at[pl.ds(e0, sk), pl.ds(p * cw, cw)],
                    msgb[b], smb[b])

        def waitld(j, b, p):
            e0 = pl.multiple_of(wbase + j * sk, 8)
            pltpu.make_async_copy(recv_hbm.at[pl.ds(e0, sk)],
                                  idxb[b], sib[b]).wait()
            if p < np_:
                pltpu.make_async_copy(
                    msgs_hbm.at[pl.ds(e0, sk), pl.ds(p * cw, cw)],
                    msgb[b], smb[b]).wait()

        for p in range(np_ + 1):
            @pl.when(jnp.logical_not(is_last))
            def _():
                pltpu.sync_copy(z128_hbm, agg_s.at[pl.ds(row0, rps)])

            @pl.when(is_last)
            def _():
                pltpu.sync_copy(z128_hbm.at[pl.ds(0, rlast)],
                                agg_s.at[pl.ds(row0, rlast)])

            plsc.subcore_barrier()

            if p == np_:
                pltpu.sync_copy(e1_hbm, msg0_v)
                pltpu.sync_copy(e1_hbm.at[pl.ds(0, rem)], msge_v)

            fire(0, 0, p)

            def super(i, carry):
                j0 = 2 * i
                fire(j0 + 1, 1, p)
                waitld(j0, 0, p)
                pltpu.sync_copy(msg0_v, agg_s.at[idx0_v], add=True)

                @pl.when(i < nch // 2 - 1)
                def _():
                    fire(j0 + 2, 0, p)

                waitld(j0 + 1, 1, p)
                pltpu.sync_copy(msg1_v if p < np_ else msg0_v,
                                agg_s.at[idx1_v], add=True)
                return carry

            lax.fori_loop(0, nch // 2, super, 0)

            if rem:
                e0r = pl.multiple_of(wbase + nch * sk, 8)
                pltpu.async_copy(recv_hbm.at[pl.ds(e0r, rem)], idxe_v,
                                 se).wait()
                if p < np_:
                    pltpu.async_copy(
                        msgs_hbm.at[pl.ds(e0r, rem), pl.ds(p * cw, cw)],
                        msge_v, se).wait()
                pltpu.sync_copy(msge_v, agg_s.at[idxe_v], add=True)
            plsc.subcore_barrier()

            @pl.when(jnp.logical_not(is_last))
            def _():
                if p < np_:
                    pltpu.sync_copy(
                        agg_s.at[pl.ds(row0, rps)],
                        agg_out.at[c, pl.ds(row0, rps), pl.ds(p * cw, cw)])
                else:
                    pltpu.sync_copy(agg_s.at[pl.ds(row0, rps)],
                                    cnt_out.at[c, pl.ds(row0, rps)])

            @pl.when(is_last)
            def _():
                if p < np_:
                    pltpu.sync_copy(
                        agg_s.at[pl.ds(row0, rlast)],
                        agg_out.at[c, pl.ds(row0, rlast), pl.ds(p * cw, cw)])
                else:
                    pltpu.sync_copy(agg_s.at[pl.ds(row0, rlast)],
                                    cnt_out.at[c, pl.ds(row0, rlast)])

            plsc.subcore_barrier()

    return k(messages, receivers, z128, e1)


def _tc_pack_nodes(nodes):
    n, d = nodes.shape
    dp = d // 2
    bn = 2000

    def body(x_ref, out_ref):
        rnd = jnp.int32(0x8000)
        left = lax.bitcast_convert_type(x_ref[:, :dp], jnp.int32)
        right = lax.bitcast_convert_type(x_ref[:, dp:], jnp.int32)
        lo = lax.shift_right_logical(left + rnd, 16)
        hi = lax.bitwise_and(right + rnd, jnp.int32(-65536))
        out_ref[...] = lax.bitwise_or(lo, hi)

    return pl.pallas_call(
        body,
        grid=(n // bn,),
        in_specs=[pl.BlockSpec((bn, d), lambda i: (i, 0))],
        out_specs=pl.BlockSpec((bn, dp), lambda i: (i, 0)),
        out_shape=jax.ShapeDtypeStruct((n, dp), jnp.int32),
    )(nodes)


def _layer_norm_in_kernel(h, g, b):
    mu = jnp.mean(h, axis=-1, keepdims=True)
    var = jnp.mean((h - mu) * (h - mu), axis=-1, keepdims=True)
    return (h - mu) * lax.rsqrt(var + 1e-6) * g + b


def _tc_edge_mlp(sfp, edges, w1lo, w1hi, w1b, b1, g1, be1, w2, b2):
    e, dp = sfp.shape
    de = edges.shape[1]
    hdim = w2.shape[1]
    be_blk = 1280
    grid = (e // be_blk,)

    def body(sf_ref, ed_ref, w1lo_ref, w1hi_ref, w1b_ref, b1_ref, g1_ref,
             be1_ref, w2_ref, b2_ref, out_ref):
        spk = sf_ref[...]
        lo = lax.bitcast_convert_type(lax.shift_left(spk, 16), jnp.float32)
        hi = lax.bitcast_convert_type(
            lax.bitwise_and(spk, jnp.int32(-65536)), jnp.float32)
        h = jnp.dot(lo, w1lo_ref[...], preferred_element_type=jnp.float32)
        h = h + jnp.dot(hi, w1hi_ref[...], preferred_element_type=jnp.float32)
        h = h + jnp.dot(ed_ref[...], w1b_ref[...],
                        preferred_element_type=jnp.float32)
        h = h + b1_ref[...]
        h = _layer_norm_in_kernel(h, g1_ref[...], be1_ref[...])
        h = jnp.maximum(h, 0.0)
        out_ref[...] = jnp.dot(h, w2_ref[...],
                               preferred_element_type=jnp.float32) + b2_ref[...]

    hsz = w1lo.shape[1]
    return pl.pallas_call(
        body,
        grid=grid,
        in_specs=[
            pl.BlockSpec((be_blk, dp), lambda i: (i, 0)),
            pl.BlockSpec((be_blk, de), lambda i: (i, 0)),
            pl.BlockSpec((dp, hsz), lambda i: (0, 0)),
            pl.BlockSpec((dp, hsz), lambda i: (0, 0)),
            pl.BlockSpec((de, hsz), lambda i: (0, 0)),
            pl.BlockSpec((1, hsz), lambda i: (0, 0)),
            pl.BlockSpec((1, hsz), lambda i: (0, 0)),
            pl.BlockSpec((1, hsz), lambda i: (0, 0)),
            pl.BlockSpec((hsz, hdim), lambda i: (0, 0)),
            pl.BlockSpec((1, hdim), lambda i: (0, 0)),
        ],
        out_specs=pl.BlockSpec((be_blk, hdim), lambda i: (i, 0)),
        out_shape=jax.ShapeDtypeStruct((e, hdim), jnp.float32),
    )(sfp, edges, w1lo, w1hi, w1b, b1, g1, be1, w2, b2)


def _tc_node_mlp(nodes, a0, a1, c0, c1, w1t, w1b, b1, g1, be1, w2, b2):
    n, d = nodes.shape
    hdim = a0.shape[1]
    do = w2.shape[1]
    bn = 1000
    grid = (n // bn,)

    def body(nd_ref, a0_ref, a1_ref, c0_ref, c1_ref, w1t_ref, w1b_ref,
             b1_ref, g1_ref, be1_ref, w2_ref, b2_ref, out_ref):
        cnt = (c0_ref[...][:, 0:1] + c1_ref[...][:, 0:1]).astype(jnp.float32)
        cnt = jnp.maximum(cnt, 1.0)
        agg = (a0_ref[...].astype(jnp.float32)
               + a1_ref[...].astype(jnp.float32)) / cnt
        h = jnp.dot(nd_ref[...].astype(jnp.bfloat16), w1t_ref[...],
                    preferred_element_type=jnp.float32)
        h = h + jnp.dot(agg.astype(jnp.bfloat16), w1b_ref[...],
                        preferred_element_type=jnp.float32)
        h = h + b1_ref[...]
        h = _layer_norm_in_kernel(h, g1_ref[...], be1_ref[...])
        h = jnp.maximum(h, 0.0)
        out = jnp.dot(h.astype(jnp.bfloat16), w2_ref[...],
                      preferred_element_type=jnp.float32) + b2_ref[...]
        out_ref[...] = out + nd_ref[...]

    hsz = w1t.shape[1]
    return pl.pallas_call(
        body,
        grid=grid,
        in_specs=[
            pl.BlockSpec((bn, d), lambda i: (i, 0)),
            pl.BlockSpec((bn, hdim), lambda i: (i, 0)),
            pl.BlockSpec((bn, hdim), lambda i: (i, 0)),
            pl.BlockSpec((bn, 128), lambda i: (i, 0)),
            pl.BlockSpec((bn, 128), lambda i: (i, 0)),
            pl.BlockSpec((d, hsz), lambda i: (0, 0)),
            pl.BlockSpec((hdim, hsz), lambda i: (0, 0)),
            pl.BlockSpec((1, hsz), lambda i: (0, 0)),
            pl.BlockSpec((1, hsz), lambda i: (0, 0)),
            pl.BlockSpec((1, hsz), lambda i: (0, 0)),
            pl.BlockSpec((hsz, do), lambda i: (0, 0)),
            pl.BlockSpec((1, do), lambda i: (0, 0)),
        ],
        out_specs=pl.BlockSpec((bn, do), lambda i: (i, 0)),
        out_shape=jax.ShapeDtypeStruct((n, do), jnp.float32),
    )(nodes, a0, a1, c0, c1, w1t, w1b, b1, g1, be1, w2, b2)


def kernel(nodes, edges, senders, receivers, W1e, b1e, g1e, be1e, W2e, b2e,
           W1n, b1n, g1n, be1n, W2n, b2n):
    n, df = nodes.shape
    e, de = edges.shape
    senders = senders.astype(jnp.int32)
    receivers = receivers.astype(jnp.int32)

    nodes_packed = _tc_pack_nodes(nodes)
    sf_packed = _sc_gather(nodes_packed, senders)
    half = df // 2
    messages = _tc_edge_mlp(
        sf_packed, edges,
        W1e[:half], W1e[half:df], W1e[df:],
        b1e[None, :], g1e[None, :], be1e[None, :],
        W2e, b2e[None, :])
    agg_part, cnt_part = _sc_scatter(messages, receivers, n)
    new_nodes = _tc_node_mlp(
        nodes, agg_part[0], agg_part[1], cnt_part[0], cnt_part[1],
        W1n[:df].astype(jnp.bfloat16), W1n[df:].astype(jnp.bfloat16),
        b1n[None, :], g1n[None, :], be1n[None, :],
        W2n.astype(jnp.bfloat16), b2n[None, :])
    return new_nodes


# speedup vs baseline: 1.9220x; 1.0449x over previous
import functools

import jax
import jax.numpy as jnp
from jax import lax
from jax.experimental import pallas as pl
from jax.experimental.pallas import tpu as pltpu
from jax.experimental.pallas import tpu_sc as plsc

_NC = 2
_NS = 16
_NW = _NC * _NS


def _sc_gather(nodes, senders):
    n, d = nodes.shape
    e = senders.shape[0]
    ew = e // _NW
    gk = 200
    nch = ew // gk
    mesh = plsc.VectorSubcoreMesh(core_axis_name="c", subcore_axis_name="s")

    @functools.partial(
        pl.kernel,
        out_type=jax.ShapeDtypeStruct((e, d), jnp.int32),
        mesh=mesh,
        scratch_types=[
            pltpu.VMEM((gk,), jnp.int32),
            pltpu.VMEM((gk,), jnp.int32),
            pltpu.VMEM((gk, d), jnp.int32),
            pltpu.VMEM((gk, d), jnp.int32),
            pltpu.SemaphoreType.DMA,
            pltpu.SemaphoreType.DMA,
            pltpu.SemaphoreType.DMA,
            pltpu.SemaphoreType.DMA,
            pltpu.SemaphoreType.DMA,
            pltpu.SemaphoreType.DMA,
        ],
    )
    def k(nodes_hbm, senders_hbm, out_hbm, idx0_v, idx1_v, rows0_v, rows1_v,
          si0, si1, sg0, sg1, sw0, sw1):
        wid = lax.axis_index("s") * _NC + lax.axis_index("c")
        idx = (idx0_v, idx1_v)
        rows = (rows0_v, rows1_v)
        si = (si0, si1)
        sg = (sg0, sg1)
        sw = (sw0, sw1)

        def base(j):
            return pl.multiple_of(wid * ew + j * gk, 8)

        def load(j):
            b = j % 2
            return pltpu.async_copy(senders_hbm.at[pl.ds(base(j), gk)],
                                    idx[b], si[b])

        def gath(j):
            b = j % 2
            return pltpu.async_copy(nodes_hbm.at[idx[b]],
                                    rows[b], sg[b])

        def wout(j):
            b = j % 2
            return pltpu.async_copy(rows[b],
                                    out_hbm.at[pl.ds(base(j), gk)], sw[b])

        dl = [None] * nch
        dg = [None] * nch
        dw = [None] * nch
        dl[0] = load(0)
        if nch > 1:
            dl[1] = load(1)
        dl[0].wait()
        dg[0] = gath(0)
        for j in range(nch):
            dg[j].wait()
            if j + 1 < nch:
                dl[j + 1].wait()
                if j >= 1:
                    dw[j - 1].wait()
                dg[j + 1] = gath(j + 1)
            dw[j] = wout(j)
            if j + 2 < nch:
                dl[j + 2] = load(j + 2)
        if nch > 1:
            dw[nch - 2].wait()
        dw[nch - 1].wait()

    return k(nodes, senders)


def _sc_scatter(messages, receivers, n):
    e, h = messages.shape
    ew = e // _NW
    sk = 192
    nch = ew // sk
    rem = ew - nch * sk
    cw = 128
    np_ = h // cw
    rps = 632
    rlast = n - (_NS - 1) * rps

    z128 = jnp.zeros((rps, cw), jnp.float32)
    e1 = jnp.zeros((sk, cw), jnp.float32).at[:, 0].set(1.0)

    mesh = plsc.VectorSubcoreMesh(core_axis_name="c", subcore_axis_name="s")

    @functools.partial(
        pl.kernel,
        out_type=(
            jax.ShapeDtypeStruct((_NC, n, h), jnp.float32),
            jax.ShapeDtypeStruct((_NC, n, cw), jnp.float32),
        ),
        mesh=mesh,
        scratch_types=[
            pltpu.VMEM((sk, cw), jnp.float32),
            pltpu.VMEM((sk, cw), jnp.float32),
            pltpu.VMEM((rem, cw), jnp.float32),
            pltpu.VMEM((sk,), jnp.int32),
            pltpu.VMEM((sk,), jnp.int32),
            pltpu.VMEM((rem,), jnp.int32),
            pltpu.VMEM_SHARED((n, cw), jnp.float32),
            pltpu.SemaphoreType.DMA,
            pltpu.SemaphoreType.DMA,
            pltpu.SemaphoreType.DMA,
            pltpu.SemaphoreType.DMA,
            pltpu.SemaphoreType.DMA,
        ],
    )
    def k(msgs_hbm, recv_hbm, z128_hbm, e1_hbm, agg_out, cnt_out,
          msg0_v, msg1_v, msge_v, idx0_v, idx1_v, idxe_v, agg_s,
          si0, si1, sm0, sm1, se):
        c = lax.axis_index("c")
        s = lax.axis_index("s")
        wid = s * _NC + c
        row0 = pl.multiple_of(s * rps, 8)
        is_last = s == _NS - 1
        wbase = wid * ew
        msgb = (msg0_v, msg1_v)
        idxb = (idx0_v, idx1_v)
        sib = (si0, si1)
        smb = (sm0, sm1)

        def fire(j, b, p):
            e0 = pl.multiple_of(wbase + j * sk, 8)
            pltpu.async_copy(recv_hbm.at[pl.ds(e0, sk)], idxb[b], sib[b])
            if p < np_:
                pltpu.async_copy(
                    msgs_hbm.at[pl.ds(e0, sk), pl.ds(p * cw, cw)],
                    msgb[b], smb[b])

        def waitld(j, b, p):
            e0 = pl.multiple_of(wbase + j * sk, 8)
            pltpu.make_async_copy(recv_hbm.at[pl.ds(e0, sk)],
                                  idxb[b], sib[b]).wait()
            if p < np_:
                pltpu.make_async_copy(
                    msgs_hbm.at[pl.ds(e0, sk), pl.ds(p * cw, cw)],
                    msgb[b], smb[b]).wait()

        for p in range(np_ + 1):
            @pl.when(jnp.logical_not(is_last))
            def _():
                pltpu.sync_copy(z128_hbm, agg_s.at[pl.ds(row0, rps)])

            @pl.when(is_last)
            def _():
                pltpu.sync_copy(z128_hbm.at[pl.ds(0, rlast)],
                                agg_s.at[pl.ds(row0, rlast)])

            plsc.subcore_barrier()

            if p == np_:
                pltpu.sync_copy(e1_hbm, msg0_v)
                pltpu.sync_copy(e1_hbm.at[pl.ds(0, rem)], msge_v)

            fire(0, 0, p)

            def super(i, carry):
                j0 = 2 * i
                fire(j0 + 1, 1, p)
                waitld(j0, 0, p)
                pltpu.sync_copy(msg0_v, agg_s.at[idx0_v], add=True)

                @pl.when(i < nch // 2 - 1)
                def _():
                    fire(j0 + 2, 0, p)

                waitld(j0 + 1, 1, p)
                pltpu.sync_copy(msg1_v if p < np_ else msg0_v,
                                agg_s.at[idx1_v], add=True)
                return carry

            lax.fori_loop(0, nch // 2, super, 0)

            if rem:
                e0r = pl.multiple_of(wbase + nch * sk, 8)
                pltpu.async_copy(recv_hbm.at[pl.ds(e0r, rem)], idxe_v,
                                 se).wait()
                if p < np_:
                    pltpu.async_copy(
                        msgs_hbm.at[pl.ds(e0r, rem), pl.ds(p * cw, cw)],
                        msge_v, se).wait()
                pltpu.sync_copy(msge_v, agg_s.at[idxe_v], add=True)
            plsc.subcore_barrier()

            @pl.when(jnp.logical_not(is_last))
            def _():
                if p < np_:
                    pltpu.sync_copy(
                        agg_s.at[pl.ds(row0, rps)],
                        agg_out.at[c, pl.ds(row0, rps), pl.ds(p * cw, cw)])
                else:
                    pltpu.sync_copy(agg_s.at[pl.ds(row0, rps)],
                                    cnt_out.at[c, pl.ds(row0, rps)])

            @pl.when(is_last)
            def _():
                if p < np_:
                    pltpu.sync_copy(
                        agg_s.at[pl.ds(row0, rlast)],
                        agg_out.at[c, pl.ds(row0, rlast), pl.ds(p * cw, cw)])
                else:
                    pltpu.sync_copy(agg_s.at[pl.ds(row0, rlast)],
                                    cnt_out.at[c, pl.ds(row0, rlast)])

            plsc.subcore_barrier()

    return k(messages, receivers, z128, e1)


def _tc_pack_nodes(nodes):
    n, d = nodes.shape
    dp = d // 2
    bn = 2000

    def body(x_ref, out_ref):
        rnd = jnp.int32(0x8000)
        left = lax.bitcast_convert_type(x_ref[:, :dp], jnp.int32)
        right = lax.bitcast_convert_type(x_ref[:, dp:], jnp.int32)
        lo = lax.shift_right_logical(left + rnd, 16)
        hi = lax.bitwise_and(right + rnd, jnp.int32(-65536))
        out_ref[...] = lax.bitwise_or(lo, hi)

    return pl.pallas_call(
        body,
        grid=(n // bn,),
        in_specs=[pl.BlockSpec((bn, d), lambda i: (i, 0))],
        out_specs=pl.BlockSpec((bn, dp), lambda i: (i, 0)),
        out_shape=jax.ShapeDtypeStruct((n, dp), jnp.int32),
    )(nodes)


def _layer_norm_in_kernel(h, g, b):
    mu = jnp.mean(h, axis=-1, keepdims=True)
    var = jnp.mean((h - mu) * (h - mu), axis=-1, keepdims=True)
    return (h - mu) * lax.rsqrt(var + 1e-6) * g + b


def _tc_edge_mlp(sfp, edges, w1lo, w1hi, w1b, b1, g1, be1, w2, b2):
    e, dp = sfp.shape
    de = edges.shape[1]
    hdim = w2.shape[1]
    be_blk = 1280
    grid = (e // be_blk,)

    def body(sf_ref, ed_ref, w1lo_ref, w1hi_ref, w1b_ref, b1_ref, g1_ref,
             be1_ref, w2_ref, b2_ref, out_ref):
        spk = sf_ref[...]
        lo = lax.bitcast_convert_type(lax.shift_left(spk, 16), jnp.float32)
        hi = lax.bitcast_convert_type(
            lax.bitwise_and(spk, jnp.int32(-65536)), jnp.float32)
        h = jnp.dot(lo, w1lo_ref[...], preferred_element_type=jnp.float32)
        h = h + jnp.dot(hi, w1hi_ref[...], preferred_element_type=jnp.float32)
        h = h + jnp.dot(ed_ref[...], w1b_ref[...],
                        preferred_element_type=jnp.float32)
        h = h + b1_ref[...]
        h = _layer_norm_in_kernel(h, g1_ref[...], be1_ref[...])
        h = jnp.maximum(h, 0.0)
        out_ref[...] = jnp.dot(h, w2_ref[...],
                               preferred_element_type=jnp.float32) + b2_ref[...]

    hsz = w1lo.shape[1]
    return pl.pallas_call(
        body,
        grid=grid,
        in_specs=[
            pl.BlockSpec((be_blk, dp), lambda i: (i, 0)),
            pl.BlockSpec((be_blk, de), lambda i: (i, 0)),
            pl.BlockSpec((dp, hsz), lambda i: (0, 0)),
            pl.BlockSpec((dp, hsz), lambda i: (0, 0)),
            pl.BlockSpec((de, hsz), lambda i: (0, 0)),
            pl.BlockSpec((1, hsz), lambda i: (0, 0)),
            pl.BlockSpec((1, hsz), lambda i: (0, 0)),
            pl.BlockSpec((1, hsz), lambda i: (0, 0)),
            pl.BlockSpec((hsz, hdim), lambda i: (0, 0)),
            pl.BlockSpec((1, hdim), lambda i: (0, 0)),
        ],
        out_specs=pl.BlockSpec((be_blk, hdim), lambda i: (i, 0)),
        out_shape=jax.ShapeDtypeStruct((e, hdim), jnp.float32),
    )(sfp, edges, w1lo, w1hi, w1b, b1, g1, be1, w2, b2)


def _tc_node_mlp(nodes, agg_part, cnt_part, w1t, w1b, b1, g1, be1, w2, b2):
    n, d = nodes.shape
    hdim = agg_part.shape[2]
    cwid = cnt_part.shape[2]
    do = w2.shape[1]
    bn = 1000
    grid = (n // bn,)

    def body(nd_ref, a_ref, c_ref, w1t_ref, w1b_ref,
             b1_ref, g1_ref, be1_ref, w2_ref, b2_ref, out_ref):
        cnt = c_ref[0, :, 0:1] + c_ref[1, :, 0:1]
        cnt = jnp.maximum(cnt, 1.0)
        agg = (a_ref[0] + a_ref[1]) / cnt
        h = jnp.dot(nd_ref[...].astype(jnp.bfloat16), w1t_ref[...],
                    preferred_element_type=jnp.float32)
        h = h + jnp.dot(agg.astype(jnp.bfloat16), w1b_ref[...],
                        preferred_element_type=jnp.float32)
        h = h + b1_ref[...]
        h = _layer_norm_in_kernel(h, g1_ref[...], be1_ref[...])
        h = jnp.maximum(h, 0.0)
        out = jnp.dot(h.astype(jnp.bfloat16), w2_ref[...],
                      preferred_element_type=jnp.float32) + b2_ref[...]
        out_ref[...] = out + nd_ref[...]

    hsz = w1t.shape[1]
    return pl.pallas_call(
        body,
        grid=grid,
        in_specs=[
            pl.BlockSpec((bn, d), lambda i: (i, 0)),
            pl.BlockSpec((2, bn, hdim), lambda i: (0, i, 0)),
            pl.BlockSpec((2, bn, cwid), lambda i: (0, i, 0)),
            pl.BlockSpec((d, hsz), lambda i: (0, 0)),
            pl.BlockSpec((hdim, hsz), lambda i: (0, 0)),
            pl.BlockSpec((1, hsz), lambda i: (0, 0)),
            pl.BlockSpec((1, hsz), lambda i: (0, 0)),
            pl.BlockSpec((1, hsz), lambda i: (0, 0)),
            pl.BlockSpec((hsz, do), lambda i: (0, 0)),
            pl.BlockSpec((1, do), lambda i: (0, 0)),
        ],
        out_specs=pl.BlockSpec((bn, do), lambda i: (i, 0)),
        out_shape=jax.ShapeDtypeStruct((n, do), jnp.float32),
    )(nodes, agg_part, cnt_part, w1t, w1b, b1, g1, be1, w2, b2)


def kernel(nodes, edges, senders, receivers, W1e, b1e, g1e, be1e, W2e, b2e,
           W1n, b1n, g1n, be1n, W2n, b2n):
    n, df = nodes.shape
    e, de = edges.shape
    senders = senders.astype(jnp.int32)
    receivers = receivers.astype(jnp.int32)

    nodes_packed = _tc_pack_nodes(nodes)
    sf_packed = _sc_gather(nodes_packed, senders)
    half = df // 2
    messages = _tc_edge_mlp(
        sf_packed, edges,
        W1e[:half], W1e[half:df], W1e[df:],
        b1e[None, :], g1e[None, :], be1e[None, :],
        W2e, b2e[None, :])
    agg_part, cnt_part = _sc_scatter(messages, receivers, n)
    new_nodes = _tc_node_mlp(
        nodes, agg_part, cnt_part,
        W1n[:df].astype(jnp.bfloat16), W1n[df:].astype(jnp.bfloat16),
        b1n[None, :], g1n[None, :], be1n[None, :],
        W2n.astype(jnp.bfloat16), b2n[None, :])
    return new_nodes


# speedup vs baseline: 1.9910x; 1.0359x over previous
import functools

import jax
import jax.numpy as jnp
from jax import lax
from jax.experimental import pallas as pl
from jax.experimental.pallas import tpu as pltpu
from jax.experimental.pallas import tpu_sc as plsc

_NC = 2
_NS = 16
_NW = _NC * _NS


def _sc_gather(nodes, senders):
    n, d = nodes.shape
    e = senders.shape[0]
    ew = e // _NW
    gk = 200
    nch = ew // gk
    mesh = plsc.VectorSubcoreMesh(core_axis_name="c", subcore_axis_name="s")

    @functools.partial(
        pl.kernel,
        out_type=jax.ShapeDtypeStruct((e, d), jnp.int32),
        mesh=mesh,
        scratch_types=[
            pltpu.VMEM((gk,), jnp.int32),
            pltpu.VMEM((gk,), jnp.int32),
            pltpu.VMEM((gk, d), jnp.int32),
            pltpu.VMEM((gk, d), jnp.int32),
            pltpu.SemaphoreType.DMA,
            pltpu.SemaphoreType.DMA,
            pltpu.SemaphoreType.DMA,
            pltpu.SemaphoreType.DMA,
            pltpu.SemaphoreType.DMA,
            pltpu.SemaphoreType.DMA,
        ],
    )
    def k(nodes_hbm, senders_hbm, out_hbm, idx0_v, idx1_v, rows0_v, rows1_v,
          si0, si1, sg0, sg1, sw0, sw1):
        wid = lax.axis_index("s") * _NC + lax.axis_index("c")
        idx = (idx0_v, idx1_v)
        rows = (rows0_v, rows1_v)
        si = (si0, si1)
        sg = (sg0, sg1)
        sw = (sw0, sw1)

        def base(j):
            return pl.multiple_of(wid * ew + j * gk, 8)

        def load(j):
            b = j % 2
            return pltpu.async_copy(senders_hbm.at[pl.ds(base(j), gk)],
                                    idx[b], si[b])

        def gath(j):
            b = j % 2
            return pltpu.async_copy(nodes_hbm.at[idx[b]],
                                    rows[b], sg[b])

        def wout(j):
            b = j % 2
            return pltpu.async_copy(rows[b],
                                    out_hbm.at[pl.ds(base(j), gk)], sw[b])

        dl = [None] * nch
        dg = [None] * nch
        dw = [None] * nch
        dl[0] = load(0)
        if nch > 1:
            dl[1] = load(1)
        dl[0].wait()
        dg[0] = gath(0)
        for j in range(nch):
            dg[j].wait()
            if j + 1 < nch:
                dl[j + 1].wait()
                if j >= 1:
                    dw[j - 1].wait()
                dg[j + 1] = gath(j + 1)
            dw[j] = wout(j)
            if j + 2 < nch:
                dl[j + 2] = load(j + 2)
        if nch > 1:
            dw[nch - 2].wait()
        dw[nch - 1].wait()

    return k(nodes, senders)


def _sc_scatter(messages, receivers, n):
    e, h = messages.shape
    ew = e // _NW
    sk = 192
    nch = ew // sk
    rem = ew - nch * sk
    cw = 128
    np_ = h // cw
    rps = 632
    rlast = n - (_NS - 1) * rps

    z128 = jnp.zeros((rps, cw), jnp.float32)
    e1 = jnp.zeros((sk, cw), jnp.float32).at[:, 0].set(1.0)

    mesh = plsc.VectorSubcoreMesh(core_axis_name="c", subcore_axis_name="s")

    @functools.partial(
        pl.kernel,
        out_type=(
            jax.ShapeDtypeStruct((_NC, n, h), jnp.float32),
            jax.ShapeDtypeStruct((_NC, n, cw), jnp.float32),
        ),
        mesh=mesh,
        scratch_types=[
            pltpu.VMEM((sk, cw), jnp.float32),
            pltpu.VMEM((sk, cw), jnp.float32),
            pltpu.VMEM((rem, cw), jnp.float32),
            pltpu.VMEM((sk,), jnp.int32),
            pltpu.VMEM((sk,), jnp.int32),
            pltpu.VMEM((rem,), jnp.int32),
            pltpu.VMEM_SHARED((n, cw), jnp.float32),
            pltpu.SemaphoreType.DMA,
            pltpu.SemaphoreType.DMA,
            pltpu.SemaphoreType.DMA,
            pltpu.SemaphoreType.DMA,
            pltpu.SemaphoreType.DMA,
        ],
    )
    def k(msgs_hbm, recv_hbm, z128_hbm, e1_hbm, agg_out, cnt_out,
          msg0_v, msg1_v, msge_v, idx0_v, idx1_v, idxe_v, agg_s,
          si0, si1, sm0, sm1, se):
        c = lax.axis_index("c")
        s = lax.axis_index("s")
        wid = s * _NC + c
        row0 = pl.multiple_of(s * rps, 8)
        is_last = s == _NS - 1
        wbase = wid * ew
        msgb = (msg0_v, msg1_v)
        idxb = (idx0_v, idx1_v)
        sib = (si0, si1)
        smb = (sm0, sm1)

        def fire(j, b, p):
            e0 = pl.multiple_of(wbase + j * sk, 8)
            pltpu.async_copy(recv_hbm.at[pl.ds(e0, sk)], idxb[b], sib[b])
            if p < np_:
                pltpu.async_copy(
                    msgs_hbm.at[pl.ds(e0, sk), pl.ds(p * cw, cw)],
                    msgb[b], smb[b])

        def waitld(j, b, p):
            e0 = pl.multiple_of(wbase + j * sk, 8)
            pltpu.make_async_copy(recv_hbm.at[pl.ds(e0, sk)],
                                  idxb[b], sib[b]).wait()
            if p < np_:
                pltpu.make_async_copy(
                    msgs_hbm.at[pl.ds(e0, sk), pl.ds(p * cw, cw)],
                    msgb[b], smb[b]).wait()

        for p in range(np_ + 1):
            @pl.when(jnp.logical_not(is_last))
            def _():
                pltpu.sync_copy(z128_hbm, agg_s.at[pl.ds(row0, rps)])

            @pl.when(is_last)
            def _():
                pltpu.sync_copy(z128_hbm.at[pl.ds(0, rlast)],
                                agg_s.at[pl.ds(row0, rlast)])

            plsc.subcore_barrier()

            if p == np_:
                pltpu.sync_copy(e1_hbm, msg0_v)
                pltpu.sync_copy(e1_hbm.at[pl.ds(0, rem)], msge_v)

            fire(0, 0, p)

            def super(i, carry):
                j0 = 2 * i
                fire(j0 + 1, 1, p)
                waitld(j0, 0, p)
                pltpu.sync_copy(msg0_v, agg_s.at[idx0_v], add=True)

                @pl.when(i < nch // 2 - 1)
                def _():
                    fire(j0 + 2, 0, p)

                waitld(j0 + 1, 1, p)
                pltpu.sync_copy(msg1_v if p < np_ else msg0_v,
                                agg_s.at[idx1_v], add=True)
                return carry

            lax.fori_loop(0, nch // 2, super, 0)

            if rem:
                e0r = pl.multiple_of(wbase + nch * sk, 8)
                pltpu.async_copy(recv_hbm.at[pl.ds(e0r, rem)], idxe_v,
                                 se).wait()
                if p < np_:
                    pltpu.async_copy(
                        msgs_hbm.at[pl.ds(e0r, rem), pl.ds(p * cw, cw)],
                        msge_v, se).wait()
                pltpu.sync_copy(msge_v, agg_s.at[idxe_v], add=True)
            plsc.subcore_barrier()

            @pl.when(jnp.logical_not(is_last))
            def _():
                if p < np_:
                    pltpu.sync_copy(
                        agg_s.at[pl.ds(row0, rps)],
                        agg_out.at[c, pl.ds(row0, rps), pl.ds(p * cw, cw)])
                else:
                    pltpu.sync_copy(agg_s.at[pl.ds(row0, rps)],
                                    cnt_out.at[c, pl.ds(row0, rps)])

            @pl.when(is_last)
            def _():
                if p < np_:
                    pltpu.sync_copy(
                        agg_s.at[pl.ds(row0, rlast)],
                        agg_out.at[c, pl.ds(row0, rlast), pl.ds(p * cw, cw)])
                else:
                    pltpu.sync_copy(agg_s.at[pl.ds(row0, rlast)],
                                    cnt_out.at[c, pl.ds(row0, rlast)])

            plsc.subcore_barrier()

    return k(messages, receivers, z128, e1)


def _tc_pack_nodes(nodes):
    n, d = nodes.shape
    dp = d // 2
    bn = 2000

    def body(x_ref, out_ref):
        rnd = jnp.int32(0x8000)
        left = lax.bitcast_convert_type(x_ref[:, :dp], jnp.int32)
        right = lax.bitcast_convert_type(x_ref[:, dp:], jnp.int32)
        lo = lax.shift_right_logical(left + rnd, 16)
        hi = lax.bitwise_and(right + rnd, jnp.int32(-65536))
        out_ref[...] = lax.bitwise_or(lo, hi)

    return pl.pallas_call(
        body,
        grid=(n // bn,),
        in_specs=[pl.BlockSpec((bn, d), lambda i: (i, 0))],
        out_specs=pl.BlockSpec((bn, dp), lambda i: (i, 0)),
        out_shape=jax.ShapeDtypeStruct((n, dp), jnp.int32),
    )(nodes)


def _layer_norm_in_kernel(h, g, b):
    mu = jnp.mean(h, axis=-1, keepdims=True)
    var = jnp.mean((h - mu) * (h - mu), axis=-1, keepdims=True)
    return (h - mu) * lax.rsqrt(var + 1e-6) * g + b


def _tc_edge_mlp(sfp, edges, w1lo, w1hi, w1b, b1, g1, be1, w2, b2):
    e, dp = sfp.shape
    de = edges.shape[1]
    hdim = w2.shape[1]
    be_blk = 1280
    grid = (e // be_blk,)

    def body(sf_ref, ed_ref, w1lo_ref, w1hi_ref, w1b_ref, b1_ref, g1_ref,
             be1_ref, w2_ref, b2_ref, out_ref):
        spk = sf_ref[...]
        lo = lax.bitcast_convert_type(lax.shift_left(spk, 16), jnp.float32)
        hi = lax.bitcast_convert_type(
            lax.bitwise_and(spk, jnp.int32(-65536)), jnp.float32)
        sf = jnp.concatenate([lo, hi], axis=1)
        w1 = jnp.concatenate([w1lo_ref[...], w1hi_ref[...]], axis=0)
        h = jnp.dot(sf, w1, preferred_element_type=jnp.float32)
        h = h + jnp.dot(ed_ref[...], w1b_ref[...],
                        preferred_element_type=jnp.float32)
        h = h + b1_ref[...]
        h = _layer_norm_in_kernel(h, g1_ref[...], be1_ref[...])
        h = jnp.maximum(h, 0.0)
        out_ref[...] = jnp.dot(h, w2_ref[...],
                               preferred_element_type=jnp.float32) + b2_ref[...]

    hsz = w1lo.shape[1]
    return pl.pallas_call(
        body,
        grid=grid,
        in_specs=[
            pl.BlockSpec((be_blk, dp), lambda i: (i, 0)),
            pl.BlockSpec((be_blk, de), lambda i: (i, 0)),
            pl.BlockSpec((dp, hsz), lambda i: (0, 0)),
            pl.BlockSpec((dp, hsz), lambda i: (0, 0)),
            pl.BlockSpec((de, hsz), lambda i: (0, 0)),
            pl.BlockSpec((1, hsz), lambda i: (0, 0)),
            pl.BlockSpec((1, hsz), lambda i: (0, 0)),
            pl.BlockSpec((1, hsz), lambda i: (0, 0)),
            pl.BlockSpec((hsz, hdim), lambda i: (0, 0)),
            pl.BlockSpec((1, hdim), lambda i: (0, 0)),
        ],
        out_specs=pl.BlockSpec((be_blk, hdim), lambda i: (i, 0)),
        out_shape=jax.ShapeDtypeStruct((e, hdim), jnp.float32),
    )(sfp, edges, w1lo, w1hi, w1b, b1, g1, be1, w2, b2)


def _tc_node_mlp(nodes, agg_part, cnt_part, w1t, w1b, b1, g1, be1, w2, b2):
    n, d = nodes.shape
    hdim = agg_part.shape[2]
    cwid = cnt_part.shape[2]
    do = w2.shape[1]
    bn = 1000
    grid = (n // bn,)

    def body(nd_ref, a_ref, c_ref, w1t_ref, w1b_ref,
             b1_ref, g1_ref, be1_ref, w2_ref, b2_ref, out_ref):
        cnt = c_ref[0, :, 0:1] + c_ref[1, :, 0:1]
        cnt = jnp.maximum(cnt, 1.0)
        agg = (a_ref[0] + a_ref[1]) / cnt
        h = jnp.dot(nd_ref[...], w1t_ref[...],
                    preferred_element_type=jnp.float32)
        h = h + jnp.dot(agg, w1b_ref[...],
                        preferred_element_type=jnp.float32)
        h = h + b1_ref[...]
        h = _layer_norm_in_kernel(h, g1_ref[...], be1_ref[...])
        h = jnp.maximum(h, 0.0)
        out = jnp.dot(h, w2_ref[...],
                      preferred_element_type=jnp.float32) + b2_ref[...]
        out_ref[...] = out + nd_ref[...]

    hsz = w1t.shape[1]
    return pl.pallas_call(
        body,
        grid=grid,
        in_specs=[
            pl.BlockSpec((bn, d), lambda i: (i, 0)),
            pl.BlockSpec((2, bn, hdim), lambda i: (0, i, 0)),
            pl.BlockSpec((2, bn, cwid), lambda i: (0, i, 0)),
            pl.BlockSpec((d, hsz), lambda i: (0, 0)),
            pl.BlockSpec((hdim, hsz), lambda i: (0, 0)),
            pl.BlockSpec((1, hsz), lambda i: (0, 0)),
            pl.BlockSpec((1, hsz), lambda i: (0, 0)),
            pl.BlockSpec((1, hsz), lambda i: (0, 0)),
            pl.BlockSpec((hsz, do), lambda i: (0, 0)),
            pl.BlockSpec((1, do), lambda i: (0, 0)),
        ],
        out_specs=pl.BlockSpec((bn, do), lambda i: (i, 0)),
        out_shape=jax.ShapeDtypeStruct((n, do), jnp.float32),
    )(nodes, agg_part, cnt_part, w1t, w1b, b1, g1, be1, w2, b2)


def kernel(nodes, edges, senders, receivers, W1e, b1e, g1e, be1e, W2e, b2e,
           W1n, b1n, g1n, be1n, W2n, b2n):
    n, df = nodes.shape
    e, de = edges.shape
    senders = senders.astype(jnp.int32)
    receivers = receivers.astype(jnp.int32)

    nodes_packed = _tc_pack_nodes(nodes)
    sf_packed = _sc_gather(nodes_packed, senders)
    half = df // 2
    messages = _tc_edge_mlp(
        sf_packed, edges,
        W1e[:half], W1e[half:df], W1e[df:],
        b1e[None, :], g1e[None, :], be1e[None, :],
        W2e, b2e[None, :])
    agg_part, cnt_part = _sc_scatter(messages, receivers, n)
    new_nodes = _tc_node_mlp(
        nodes, agg_part, cnt_part,
        W1n[:df], W1n[df:],
        b1n[None, :], g1n[None, :], be1n[None, :],
        W2n, b2n[None, :])
    return new_nodes


# speedup vs baseline: 2.0940x; 1.0517x over previous
import functools

import jax
import jax.numpy as jnp
from jax import lax
from jax.experimental import pallas as pl
from jax.experimental.pallas import tpu as pltpu
from jax.experimental.pallas import tpu_sc as plsc

_NC = 2
_NS = 16
_NW = _NC * _NS


def _sc_gather(nodes, senders):
    n, d = nodes.shape
    e = senders.shape[0]
    ew = e // _NW
    gk = 200
    nch = ew // gk
    mesh = plsc.VectorSubcoreMesh(core_axis_name="c", subcore_axis_name="s")

    @functools.partial(
        pl.kernel,
        out_type=jax.ShapeDtypeStruct((e, d), jnp.int32),
        mesh=mesh,
        scratch_types=[
            pltpu.VMEM((gk,), jnp.int32),
            pltpu.VMEM((gk,), jnp.int32),
            pltpu.VMEM((gk, d), jnp.int32),
            pltpu.VMEM((gk, d), jnp.int32),
            pltpu.SemaphoreType.DMA,
            pltpu.SemaphoreType.DMA,
            pltpu.SemaphoreType.DMA,
            pltpu.SemaphoreType.DMA,
            pltpu.SemaphoreType.DMA,
            pltpu.SemaphoreType.DMA,
        ],
    )
    def k(nodes_hbm, senders_hbm, out_hbm, idx0_v, idx1_v, rows0_v, rows1_v,
          si0, si1, sg0, sg1, sw0, sw1):
        wid = lax.axis_index("s") * _NC + lax.axis_index("c")
        idx = (idx0_v, idx1_v)
        rows = (rows0_v, rows1_v)
        si = (si0, si1)
        sg = (sg0, sg1)
        sw = (sw0, sw1)

        def base(j):
            return pl.multiple_of(wid * ew + j * gk, 8)

        def load(j):
            b = j % 2
            return pltpu.async_copy(senders_hbm.at[pl.ds(base(j), gk)],
                                    idx[b], si[b])

        def gath(j):
            b = j % 2
            return pltpu.async_copy(nodes_hbm.at[idx[b]],
                                    rows[b], sg[b])

        def wout(j):
            b = j % 2
            return pltpu.async_copy(rows[b],
                                    out_hbm.at[pl.ds(base(j), gk)], sw[b])

        dl = [None] * nch
        dg = [None] * nch
        dw = [None] * nch
        dl[0] = load(0)
        if nch > 1:
            dl[1] = load(1)
        dl[0].wait()
        dg[0] = gath(0)
        for j in range(nch):
            dg[j].wait()
            if j + 1 < nch:
                dl[j + 1].wait()
                if j >= 1:
                    dw[j - 1].wait()
                dg[j + 1] = gath(j + 1)
            dw[j] = wout(j)
            if j + 2 < nch:
                dl[j + 2] = load(j + 2)
        if nch > 1:
            dw[nch - 2].wait()
        dw[nch - 1].wait()

    return k(nodes, senders)


def _sc_scatter(messages, receivers, n):
    e, h = messages.shape
    ew = e // _NW
    sk = 192
    nch = ew // sk
    rem = ew - nch * sk
    cw = 128
    np_ = h // cw
    rps = 632
    rlast = n - (_NS - 1) * rps

    z128 = jnp.zeros((rps, cw), jnp.float32)
    e1 = jnp.zeros((sk, cw), jnp.float32).at[:, 0].set(1.0)

    mesh = plsc.VectorSubcoreMesh(core_axis_name="c", subcore_axis_name="s")

    @functools.partial(
        pl.kernel,
        out_type=(
            jax.ShapeDtypeStruct((_NC, n, h), jnp.float32),
            jax.ShapeDtypeStruct((_NC, n, cw), jnp.float32),
        ),
        mesh=mesh,
        scratch_types=[
            pltpu.VMEM((sk, cw), jnp.float32),
            pltpu.VMEM((sk, cw), jnp.float32),
            pltpu.VMEM((rem, cw), jnp.float32),
            pltpu.VMEM((sk,), jnp.int32),
            pltpu.VMEM((sk,), jnp.int32),
            pltpu.VMEM((rem,), jnp.int32),
            pltpu.VMEM_SHARED((n, cw), jnp.float32),
            pltpu.SemaphoreType.DMA,
            pltpu.SemaphoreType.DMA,
            pltpu.SemaphoreType.DMA,
            pltpu.SemaphoreType.DMA,
            pltpu.SemaphoreType.DMA,
        ],
    )
    def k(msgs_hbm, recv_hbm, z128_hbm, e1_hbm, agg_out, cnt_out,
          msg0_v, msg1_v, msge_v, idx0_v, idx1_v, idxe_v, agg_s,
          si0, si1, sm0, sm1, se):
        c = lax.axis_index("c")
        s = lax.axis_index("s")
        wid = s * _NC + c
        row0 = pl.multiple_of(s * rps, 8)
        is_last = s == _NS - 1
        wbase = wid * ew
        msgb = (msg0_v, msg1_v)
        idxb = (idx0_v, idx1_v)
        sib = (si0, si1)
        smb = (sm0, sm1)

        def fire(j, b, p):
            e0 = pl.multiple_of(wbase + j * sk, 8)
            pltpu.async_copy(recv_hbm.at[pl.ds(e0, sk)], idxb[b], sib[b])
            if p < np_:
                pltpu.async_copy(
                    msgs_hbm.at[pl.ds(e0, sk), pl.ds(p * cw, cw)],
                    msgb[b], smb[b])

        def waitld(j, b, p):
            e0 = pl.multiple_of(wbase + j * sk, 8)
            pltpu.make_async_copy(recv_hbm.at[pl.ds(e0, sk)],
                                  idxb[b], sib[b]).wait()
            if p < np_:
                pltpu.make_async_copy(
                    msgs_hbm.at[pl.ds(e0, sk), pl.ds(p * cw, cw)],
                    msgb[b], smb[b]).wait()

        for p in range(np_ + 1):
            @pl.when(jnp.logical_not(is_last))
            def _():
                pltpu.sync_copy(z128_hbm, agg_s.at[pl.ds(row0, rps)])

            @pl.when(is_last)
            def _():
                pltpu.sync_copy(z128_hbm.at[pl.ds(0, rlast)],
                                agg_s.at[pl.ds(row0, rlast)])

            plsc.subcore_barrier()

            if p == np_:
                pltpu.sync_copy(e1_hbm, msg0_v)
                pltpu.sync_copy(e1_hbm.at[pl.ds(0, rem)], msge_v)

            fire(0, 0, p)

            def super(i, carry):
                j0 = 2 * i
                fire(j0 + 1, 1, p)
                waitld(j0, 0, p)
                pltpu.sync_copy(msg0_v, agg_s.at[idx0_v], add=True)

                @pl.when(i < nch // 2 - 1)
                def _():
                    fire(j0 + 2, 0, p)

                waitld(j0 + 1, 1, p)
                pltpu.sync_copy(msg1_v if p < np_ else msg0_v,
                                agg_s.at[idx1_v], add=True)
                return carry

            lax.fori_loop(0, nch // 2, super, 0)

            if rem:
                e0r = pl.multiple_of(wbase + nch * sk, 8)
                pltpu.async_copy(recv_hbm.at[pl.ds(e0r, rem)], idxe_v,
                                 se).wait()
                if p < np_:
                    pltpu.async_copy(
                        msgs_hbm.at[pl.ds(e0r, rem), pl.ds(p * cw, cw)],
                        msge_v, se).wait()
                pltpu.sync_copy(msge_v, agg_s.at[idxe_v], add=True)
            plsc.subcore_barrier()

            @pl.when(jnp.logical_not(is_last))
            def _():
                if p < np_:
                    pltpu.sync_copy(
                        agg_s.at[pl.ds(row0, rps)],
                        agg_out.at[c, pl.ds(row0, rps), pl.ds(p * cw, cw)])
                else:
                    pltpu.sync_copy(agg_s.at[pl.ds(row0, rps)],
                                    cnt_out.at[c, pl.ds(row0, rps)])

            @pl.when(is_last)
            def _():
                if p < np_:
                    pltpu.sync_copy(
                        agg_s.at[pl.ds(row0, rlast)],
                        agg_out.at[c, pl.ds(row0, rlast), pl.ds(p * cw, cw)])
                else:
                    pltpu.sync_copy(agg_s.at[pl.ds(row0, rlast)],
                                    cnt_out.at[c, pl.ds(row0, rlast)])

            plsc.subcore_barrier()

    return k(messages, receivers, z128, e1)


def _tc_pack_nodes(nodes):
    n, d = nodes.shape
    dp = d // 2
    bn = 2000

    def body(x_ref, out_ref):
        rnd = jnp.int32(0x8000)
        left = lax.bitcast_convert_type(x_ref[:, :dp], jnp.int32)
        right = lax.bitcast_convert_type(x_ref[:, dp:], jnp.int32)
        lo = lax.shift_right_logical(left + rnd, 16)
        hi = lax.bitwise_and(right + rnd, jnp.int32(-65536))
        out_ref[...] = lax.bitwise_or(lo, hi)

    return pl.pallas_call(
        body,
        grid=(n // bn,),
        in_specs=[pl.BlockSpec((bn, d), lambda i: (i, 0))],
        out_specs=pl.BlockSpec((bn, dp), lambda i: (i, 0)),
        out_shape=jax.ShapeDtypeStruct((n, dp), jnp.int32),
    )(nodes)


def _layer_norm_in_kernel(h, g, b):
    mu = jnp.mean(h, axis=-1, keepdims=True)
    var = jnp.mean((h - mu) * (h - mu), axis=-1, keepdims=True)
    return (h - mu) * lax.rsqrt(var + 1e-6) * g + b


def _tc_edge_mlp(sfp, edges, w1lo, w1hi, w1b, b1, g1, be1, w2, b2):
    e, dp = sfp.shape
    de = edges.shape[1]
    hdim = w2.shape[1]
    be_blk = 2560
    grid = (e // be_blk,)

    def body(sf_ref, ed_ref, w1lo_ref, w1hi_ref, w1b_ref, b1_ref, g1_ref,
             be1_ref, w2_ref, b2_ref, out_ref):
        spk = sf_ref[...]
        lo = lax.bitcast_convert_type(lax.shift_left(spk, 16), jnp.float32)
        hi = lax.bitcast_convert_type(
            lax.bitwise_and(spk, jnp.int32(-65536)), jnp.float32)
        sf = jnp.concatenate([lo, hi], axis=1)
        w1 = jnp.concatenate([w1lo_ref[...], w1hi_ref[...]], axis=0)
        h = jnp.dot(sf, w1, preferred_element_type=jnp.float32)
        h = h + jnp.dot(ed_ref[...], w1b_ref[...],
                        preferred_element_type=jnp.float32)
        h = h + b1_ref[...]
        h = _layer_norm_in_kernel(h, g1_ref[...], be1_ref[...])
        h = jnp.maximum(h, 0.0)
        out_ref[...] = jnp.dot(h, w2_ref[...],
                               preferred_element_type=jnp.float32) + b2_ref[...]

    hsz = w1lo.shape[1]
    return pl.pallas_call(
        body,
        grid=grid,
        in_specs=[
            pl.BlockSpec((be_blk, dp), lambda i: (i, 0)),
            pl.BlockSpec((be_blk, de), lambda i: (i, 0)),
            pl.BlockSpec((dp, hsz), lambda i: (0, 0)),
            pl.BlockSpec((dp, hsz), lambda i: (0, 0)),
            pl.BlockSpec((de, hsz), lambda i: (0, 0)),
            pl.BlockSpec((1, hsz), lambda i: (0, 0)),
            pl.BlockSpec((1, hsz), lambda i: (0, 0)),
            pl.BlockSpec((1, hsz), lambda i: (0, 0)),
            pl.BlockSpec((hsz, hdim), lambda i: (0, 0)),
            pl.BlockSpec((1, hdim), lambda i: (0, 0)),
        ],
        out_specs=pl.BlockSpec((be_blk, hdim), lambda i: (i, 0)),
        out_shape=jax.ShapeDtypeStruct((e, hdim), jnp.float32),
    )(sfp, edges, w1lo, w1hi, w1b, b1, g1, be1, w2, b2)


def _tc_node_mlp(nodes, agg_part, cnt_part, w1t, w1b, b1, g1, be1, w2, b2):
    n, d = nodes.shape
    hdim = agg_part.shape[2]
    cwid = cnt_part.shape[2]
    do = w2.shape[1]
    bn = 1000
    grid = (n // bn,)

    def body(nd_ref, a_ref, c_ref, w1t_ref, w1b_ref,
             b1_ref, g1_ref, be1_ref, w2_ref, b2_ref, out_ref):
        cnt = c_ref[0, :, 0:1] + c_ref[1, :, 0:1]
        cnt = jnp.maximum(cnt, 1.0)
        agg = (a_ref[0] + a_ref[1]) / cnt
        h = jnp.dot(nd_ref[...], w1t_ref[...],
                    preferred_element_type=jnp.float32)
        h = h + jnp.dot(agg, w1b_ref[...],
                        preferred_element_type=jnp.float32)
        h = h + b1_ref[...]
        h = _layer_norm_in_kernel(h, g1_ref[...], be1_ref[...])
        h = jnp.maximum(h, 0.0)
        out = jnp.dot(h, w2_ref[...],
                      preferred_element_type=jnp.float32) + b2_ref[...]
        out_ref[...] = out + nd_ref[...]

    hsz = w1t.shape[1]
    return pl.pallas_call(
        body,
        grid=grid,
        in_specs=[
            pl.BlockSpec((bn, d), lambda i: (i, 0)),
            pl.BlockSpec((2, bn, hdim), lambda i: (0, i, 0)),
            pl.BlockSpec((2, bn, cwid), lambda i: (0, i, 0)),
            pl.BlockSpec((d, hsz), lambda i: (0, 0)),
            pl.BlockSpec((hdim, hsz), lambda i: (0, 0)),
            pl.BlockSpec((1, hsz), lambda i: (0, 0)),
            pl.BlockSpec((1, hsz), lambda i: (0, 0)),
            pl.BlockSpec((1, hsz), lambda i: (0, 0)),
            pl.BlockSpec((hsz, do), lambda i: (0, 0)),
            pl.BlockSpec((1, do), lambda i: (0, 0)),
        ],
        out_specs=pl.BlockSpec((bn, do), lambda i: (i, 0)),
        out_shape=jax.ShapeDtypeStruct((n, do), jnp.float32),
    )(nodes, agg_part, cnt_part, w1t, w1b, b1, g1, be1, w2, b2)


def kernel(nodes, edges, senders, receivers, W1e, b1e, g1e, be1e, W2e, b2e,
           W1n, b1n, g1n, be1n, W2n, b2n):
    n, df = nodes.shape
    e, de = edges.shape
    senders = senders.astype(jnp.int32)
    receivers = receivers.astype(jnp.int32)

    nodes_packed = _tc_pack_nodes(nodes)
    sf_packed = _sc_gather(nodes_packed, senders)
    half = df // 2
    messages = _tc_edge_mlp(
        sf_packed, edges,
        W1e[:half], W1e[half:df], W1e[df:],
        b1e[None, :], g1e[None, :], be1e[None, :],
        W2e, b2e[None, :])
    agg_part, cnt_part = _sc_scatter(messages, receivers, n)
    new_nodes = _tc_node_mlp(
        nodes, agg_part, cnt_part,
        W1n[:df], W1n[df:],
        b1n[None, :], g1n[None, :], be1n[None, :],
        W2n, b2n[None, :])
    return new_nodes
